# Initial kernel scaffold; baseline (speedup 1.0000x reference)
import functools

import jax
import jax.numpy as jnp
from jax import lax
from jax.experimental import pallas as pl
from jax.experimental.pallas import tpu as pltpu
from jax.experimental.pallas import tpu_sc as plsc

N = 10000
E = 320000
H = 128
NC = 2
NS = 16
L = 16
NW = NC * NS
EPW = E // NW
K = 80
CH = EPW // K
RPT = N // NS
NJ = H // L


def _prep_body(emb_ref, feat_ref, w1_ref, w2_ref, bl_ref, a_ref, b_ref):
    a_ref[...] = emb_ref[...] @ w1_ref[...]
    b_ref[...] = feat_ref[...] @ w2_ref[...] + bl_ref[...]


def _tc_prep(emb, feat, w1t, w2t, bl):
    bn = 500
    grid = (N // bn,)
    return pl.pallas_call(
        _prep_body,
        grid=grid,
        in_specs=[
            pl.BlockSpec((bn, H), lambda i: (i, 0)),
            pl.BlockSpec((bn, H), lambda i: (i, 0)),
            pl.BlockSpec((H, H), lambda i: (0, 0)),
            pl.BlockSpec((H, H), lambda i: (0, 0)),
            pl.BlockSpec((1, H), lambda i: (0, 0)),
        ],
        out_specs=[
            pl.BlockSpec((bn, H), lambda i: (i, 0)),
            pl.BlockSpec((bn, H), lambda i: (i, 0)),
        ],
        out_shape=[
            jax.ShapeDtypeStruct((N, H), jnp.float32),
            jax.ShapeDtypeStruct((N, H), jnp.float32),
        ],
    )(emb, feat, w1t, w2t, bl)


def _update_body(a0_ref, s0_ref, s1_ref, d0_ref, d1_ref, a1_ref):
    deg = d0_ref[...] + d1_ref[...]
    d = deg[:, 0:1]
    ssum = s0_ref[...] + s1_ref[...]
    a1_ref[...] = jnp.where(d > 0.0, ssum / jnp.maximum(d, 1.0), a0_ref[...])


def _tc_update(a0, s0, s1, d0, d1):
    bn = 500
    grid = (N // bn,)
    return pl.pallas_call(
        _update_body,
        grid=grid,
        in_specs=[
            pl.BlockSpec((bn, H), lambda i: (i, 0)),
            pl.BlockSpec((bn, H), lambda i: (i, 0)),
            pl.BlockSpec((bn, H), lambda i: (i, 0)),
            pl.BlockSpec((bn, L), lambda i: (i, 0)),
            pl.BlockSpec((bn, L), lambda i: (i, 0)),
        ],
        out_specs=pl.BlockSpec((bn, H), lambda i: (i, 0)),
        out_shape=jax.ShapeDtypeStruct((N, H), jnp.float32),
    )(a0, s0, s1, d0, d1)


def _sc_mesh():
    return plsc.VectorSubcoreMesh(
        core_axis_name="c", subcore_axis_name="s", num_cores=NC, num_subcores=NS
    )


def _edge_dot(a_v, b_v, wu_v, s_v):
    @pl.loop(0, K)
    def _(e):
        acc = jnp.zeros((L,), jnp.float32)
        for j in range(NJ):
            av = a_v[e, pl.ds(j * L, L)]
            bv = b_v[e, pl.ds(j * L, L)]
            t = jnp.maximum(av + bv, 0.0)
            acc = acc + t * wu_v[pl.ds(j * L, L)]
        s_v[e] = jnp.sum(acc)


def _sc_block1(a_hbm, b_hbm, src, dst, wu, z128, z16, ones):
    @functools.partial(
        pl.kernel,
        out_type=[
            jax.ShapeDtypeStruct((E,), jnp.float32),
            jax.ShapeDtypeStruct((NC, N, H), jnp.float32),
            jax.ShapeDtypeStruct((NC, N, L), jnp.float32),
        ],
        mesh=_sc_mesh(),
        scratch_types=[
            pltpu.VMEM((K,), jnp.int32),
            pltpu.VMEM((K,), jnp.int32),
            pltpu.VMEM((K, H), jnp.float32),
            pltpu.VMEM((K, H), jnp.float32),
            pltpu.VMEM((K,), jnp.float32),
            pltpu.VMEM((H,), jnp.float32),
            pltpu.VMEM((K, L), jnp.float32),
            pltpu.VMEM_SHARED((N, H), jnp.float32),
            pltpu.VMEM_SHARED((N, L), jnp.float32),
            pltpu.SemaphoreType.DMA,
            pltpu.SemaphoreType.DMA,
        ],
    )
    def k(a_ref, b_ref, src_ref, dst_ref, wu_ref, z128_ref, z16_ref, ones_ref,
          s_out, asum_out, deg_out,
          idx_s, idx_d, a_v, b_v, s_v, wu_v, ones_v, asum_sh, deg_sh,
          sem_a, sem_b):
        c = lax.axis_index("c")
        sid = lax.axis_index("s")
        wid = sid * NC + c
        base = wid * EPW
        r0 = sid * RPT

        pltpu.sync_copy(wu_ref, wu_v)
        pltpu.sync_copy(ones_ref, ones_v)
        pltpu.sync_copy(z128_ref.at[pl.ds(r0, RPT)], asum_sh.at[pl.ds(r0, RPT)])
        pltpu.sync_copy(z16_ref.at[pl.ds(r0, RPT)], deg_sh.at[pl.ds(r0, RPT)])
        plsc.subcore_barrier()

        @pl.loop(0, CH)
        def _(ci):
            off = base + ci * K
            pltpu.sync_copy(src_ref.at[pl.ds(off, K)], idx_s)
            pltpu.sync_copy(dst_ref.at[pl.ds(off, K)], idx_d)
            cp_a = pltpu.async_copy(a_ref.at[idx_s], a_v, sem_a)
            cp_b = pltpu.async_copy(b_ref.at[idx_d], b_v, sem_b)
            cp_a.wait()
            cp_b.wait()
            _edge_dot(a_v, b_v, wu_v, s_v)
            pltpu.sync_copy(s_v, s_out.at[pl.ds(off, K)])
            pltpu.sync_copy(a_v, asum_sh.at[idx_d], add=True)
            pltpu.sync_copy(ones_v, deg_sh.at[idx_d], add=True)

        plsc.subcore_barrier()
        pltpu.sync_copy(asum_sh.at[pl.ds(r0, RPT)], asum_out.at[c, pl.ds(r0, RPT)])
        pltpu.sync_copy(deg_sh.at[pl.ds(r0, RPT)], deg_out.at[c, pl.ds(r0, RPT)])

    return k(a_hbm, b_hbm, src, dst, wu, z128, z16, ones)


def _sc_block2(a_hbm, b_hbm, src, dst, wu):
    @functools.partial(
        pl.kernel,
        out_type=jax.ShapeDtypeStruct((E,), jnp.float32),
        mesh=_sc_mesh(),
        scratch_types=[
            pltpu.VMEM((K,), jnp.int32),
            pltpu.VMEM((K,), jnp.int32),
            pltpu.VMEM((K, H), jnp.float32),
            pltpu.VMEM((K, H), jnp.float32),
            pltpu.VMEM((K,), jnp.float32),
            pltpu.VMEM((H,), jnp.float32),
            pltpu.SemaphoreType.DMA,
            pltpu.SemaphoreType.DMA,
        ],
    )
    def k(a_ref, b_ref, src_ref, dst_ref, wu_ref,
          s_out, idx_s, idx_d, a_v, b_v, s_v, wu_v, sem_a, sem_b):
        c = lax.axis_index("c")
        sid = lax.axis_index("s")
        wid = sid * NC + c
        base = wid * EPW

        pltpu.sync_copy(wu_ref, wu_v)

        @pl.loop(0, CH)
        def _(ci):
            off = base + ci * K
            pltpu.sync_copy(src_ref.at[pl.ds(off, K)], idx_s)
            pltpu.sync_copy(dst_ref.at[pl.ds(off, K)], idx_d)
            cp_a = pltpu.async_copy(a_ref.at[idx_s], a_v, sem_a)
            cp_b = pltpu.async_copy(b_ref.at[idx_d], b_v, sem_b)
            cp_a.wait()
            cp_b.wait()
            _edge_dot(a_v, b_v, wu_v, s_v)
            pltpu.sync_copy(s_v, s_out.at[pl.ds(off, K)])

    return k(a_hbm, b_hbm, src, dst, wu)


def kernel(features, emb, edge_index, W_z, b_z, W_l, b_l, W_u, b_u):
    src = edge_index[0]
    dst = edge_index[1]
    w1t = W_l[:, :H].T
    w2t = W_l[:, H:].T
    bl = b_l.reshape(1, H)
    wu = W_u[0]

    a0, b_tab = _tc_prep(emb, features, w1t, w2t, bl)

    z128 = jnp.zeros((N, H), jnp.float32)
    z16 = jnp.zeros((N, L), jnp.float32)
    ones = jnp.ones((K, L), jnp.float32)

    s1, asum, deg = _sc_block1(a0, b_tab, src, dst, wu, z128, z16, ones)
    a1 = _tc_update(a0, asum[0], asum[1], deg[0], deg[1])
    s2 = _sc_block2(a1, b_tab, src, dst, wu)

    s1 = s1.reshape(E, 1) + b_u[0]
    s2 = s2.reshape(E, 1) + b_u[0]
    return (s1, s2)


# speedup vs baseline: 2.0124x; 2.0124x over previous
import dataclasses
import functools

import jax
import jax.numpy as jnp
from jax import lax
from jax.experimental import pallas as pl
from jax.experimental.pallas import tpu as pltpu
from jax.experimental.pallas import tpu_sc as plsc

N = 10000
E = 320000
H = 128
NC = 2
NS = 16
L = 16
NW = NC * NS
EPW = E // NW
K = 40
CH = EPW // K
NP = 10240
RPS = NP // NS
NJ = H // L


def _prep_body(emb_ref, feat_ref, w1_ref, w2_ref, bl_ref, a_ref, b_ref):
    a_ref[...] = emb_ref[...] @ w1_ref[...]
    b_ref[...] = feat_ref[...] @ w2_ref[...] + bl_ref[...]


def _tc_prep(emb, feat, w1t, w2t, bl):
    bn = 1000
    grid = (N // bn,)
    return pl.pallas_call(
        _prep_body,
        grid=grid,
        in_specs=[
            pl.BlockSpec((bn, H), lambda i: (i, 0)),
            pl.BlockSpec((bn, H), lambda i: (i, 0)),
            pl.BlockSpec((H, H), lambda i: (0, 0)),
            pl.BlockSpec((H, H), lambda i: (0, 0)),
            pl.BlockSpec((1, H), lambda i: (0, 0)),
        ],
        out_specs=[
            pl.BlockSpec((bn, H), lambda i: (i, 0)),
            pl.BlockSpec((bn, H), lambda i: (i, 0)),
        ],
        out_shape=[
            jax.ShapeDtypeStruct((N, H), jnp.float32),
            jax.ShapeDtypeStruct((N, H), jnp.float32),
        ],
    )(emb, feat, w1t, w2t, bl)


def _update_body(a0_ref, s0_ref, s1_ref, dp_ref, a1_ref):
    d = jnp.sum(dp_ref[...], axis=1, keepdims=True)
    ssum = s0_ref[...] + s1_ref[...]
    a1_ref[...] = jnp.where(d > 0.0, ssum / jnp.maximum(d, 1.0), a0_ref[...])


def _tc_update(a0, s0, s1, dparts_t):
    bn = 1000
    grid = (N // bn,)
    return pl.pallas_call(
        _update_body,
        grid=grid,
        in_specs=[
            pl.BlockSpec((bn, H), lambda i: (i, 0)),
            pl.BlockSpec((bn, H), lambda i: (i, 0)),
            pl.BlockSpec((bn, H), lambda i: (i, 0)),
            pl.BlockSpec((bn, NW), lambda i: (i, 0)),
        ],
        out_specs=pl.BlockSpec((bn, H), lambda i: (i, 0)),
        out_shape=jax.ShapeDtypeStruct((N, H), jnp.float32),
    )(a0, s0, s1, dparts_t)


def _finish_body(p1_ref, p2_ref, bu_ref, s1_ref, s2_ref):
    bu = bu_ref[0, 0]
    s1_ref[...] = jnp.sum(p1_ref[...], axis=1, keepdims=True) + bu
    s2_ref[...] = jnp.sum(p2_ref[...], axis=1, keepdims=True) + bu


def _tc_finish(p1, p2, bu):
    bn = 8000
    grid = (E // bn,)
    return pl.pallas_call(
        _finish_body,
        grid=grid,
        in_specs=[
            pl.BlockSpec((bn, L), lambda i: (i, 0)),
            pl.BlockSpec((bn, L), lambda i: (i, 0)),
            pl.BlockSpec((1, 1), lambda i: (0, 0)),
        ],
        out_specs=[
            pl.BlockSpec((bn, 1), lambda i: (i, 0)),
            pl.BlockSpec((bn, 1), lambda i: (i, 0)),
        ],
        out_shape=[
            jax.ShapeDtypeStruct((E, 1), jnp.float32),
            jax.ShapeDtypeStruct((E, 1), jnp.float32),
        ],
    )(p1, p2, bu)


def _sc_mesh():
    return plsc.VectorSubcoreMesh(
        core_axis_name="c", subcore_axis_name="s", num_cores=NC, num_subcores=NS
    )


def _sc_params():
    cp = pltpu.CompilerParams()
    if "needs_layout_passes" in pltpu.CompilerParams.__dataclass_fields__:
        cp = dataclasses.replace(cp, needs_layout_passes=False)
    return cp


def _edge_dot(a_v, b_v, wu_v, p_v):
    @pl.loop(0, K)
    def _(e):
        acc = jnp.zeros((L,), jnp.float32)
        for j in range(NJ):
            av = a_v[e, pl.ds(j * L, L)]
            bv = b_v[e, pl.ds(j * L, L)]
            t = jnp.maximum(av + bv, 0.0)
            acc = acc + t * wu_v[pl.ds(j * L, L)]
        p_v[e, :] = acc


def _count_deg(deg_l, idx_d, ovh_v):
    ov = jnp.ones((L,), jnp.float32)
    plsc.addupdate_scatter(deg_l, [idx_d[0, pl.ds(0, L)]], ov)
    plsc.addupdate_scatter(deg_l, [idx_d[0, pl.ds(L, L)]], ov)
    plsc.addupdate_scatter(deg_l, [idx_d[0, pl.ds(K - L, L)]], ovh_v[...])


def _sc_block1(a_hbm, b_hbm, src, dst, wu, ovh):
    @functools.partial(
        pl.kernel,
        out_type=[
            jax.ShapeDtypeStruct((E, L), jnp.float32),
            jax.ShapeDtypeStruct((NC, NP, H), jnp.float32),
            jax.ShapeDtypeStruct((NC, NS, NP), jnp.float32),
        ],
        mesh=_sc_mesh(),
        compiler_params=_sc_params(),
        scratch_types=[
            pltpu.VMEM((K,), jnp.int32),
            pltpu.VMEM((1, K), jnp.int32),
            pltpu.VMEM((K,), jnp.int32),
            pltpu.VMEM((1, K), jnp.int32),
            pltpu.VMEM((K, H), jnp.float32),
            pltpu.VMEM((K, H), jnp.float32),
            pltpu.VMEM((K, H), jnp.float32),
            pltpu.VMEM((K, H), jnp.float32),
            pltpu.VMEM((K, L), jnp.float32),
            pltpu.VMEM((H,), jnp.float32),
            pltpu.VMEM((L,), jnp.float32),
            pltpu.VMEM((NP,), jnp.float32),
            pltpu.VMEM_SHARED((NP, H), jnp.float32),
            pltpu.SemaphoreType.DMA,
            pltpu.SemaphoreType.DMA,
            pltpu.SemaphoreType.DMA,
            pltpu.SemaphoreType.DMA,
        ],
    )
    def k(a_ref, b_ref, src_ref, dst_ref, wu_ref, ovh_ref,
          p_out, asum_out, deg_out,
          idx_sa, idx_da, idx_sb, idx_db, a_va, b_va, a_vb, b_vb,
          p_v, wu_v, ovh_v, deg_l, asum_sh,
          sem_a, sem_b, sem_c, sem_d):
        c = lax.axis_index("c")
        sid = lax.axis_index("s")
        wid = sid * NC + c
        base = wid * EPW
        r0 = sid * RPS

        pltpu.sync_copy(wu_ref, wu_v)
        pltpu.sync_copy(ovh_ref, ovh_v)

        zv = jnp.zeros((L,), jnp.float32)

        @pl.loop(0, K)
        def _(e):
            for j in range(NJ):
                a_va[e, pl.ds(j * L, L)] = zv

        @pl.loop(0, NP // L)
        def _(t):
            deg_l[pl.ds(t * L, L)] = zv

        @pl.loop(0, RPS // K)
        def _(t):
            pltpu.sync_copy(a_va, asum_sh.at[pl.ds(r0 + t * K, K)])

        plsc.subcore_barrier()

        @pl.loop(0, CH // 2)
        def _(g):
            offa = base + 2 * g * K
            offb = offa + K
            pltpu.sync_copy(src_ref.at[pl.ds(offa, K)], idx_sa)
            pltpu.sync_copy(dst_ref.at[pl.ds(offa, K)], idx_da.at[0])
            cpa1 = pltpu.async_copy(a_ref.at[idx_sa], a_va, sem_a)
            cpa2 = pltpu.async_copy(b_ref.at[idx_da.at[0]], b_va, sem_b)
            pltpu.sync_copy(src_ref.at[pl.ds(offb, K)], idx_sb)
            pltpu.sync_copy(dst_ref.at[pl.ds(offb, K)], idx_db.at[0])
            cpb1 = pltpu.async_copy(a_ref.at[idx_sb], a_vb, sem_c)
            cpb2 = pltpu.async_copy(b_ref.at[idx_db.at[0]], b_vb, sem_d)
            cpa1.wait()
            cpa2.wait()
            _edge_dot(a_va, b_va, wu_v, p_v)
            pltpu.sync_copy(p_v, p_out.at[pl.ds(offa, K)])
            pltpu.sync_copy(a_va, asum_sh.at[idx_da.at[0]], add=True)
            _count_deg(deg_l, idx_da, ovh_v)
            cpb1.wait()
            cpb2.wait()
            _edge_dot(a_vb, b_vb, wu_v, p_v)
            pltpu.sync_copy(p_v, p_out.at[pl.ds(offb, K)])
            pltpu.sync_copy(a_vb, asum_sh.at[idx_db.at[0]], add=True)
            _count_deg(deg_l, idx_db, ovh_v)

        plsc.subcore_barrier()

        @pl.loop(0, RPS // K)
        def _(t):
            q0 = r0 + t * K
            pltpu.sync_copy(asum_sh.at[pl.ds(q0, K)], a_va)
            pltpu.sync_copy(a_va, asum_out.at[c, pl.ds(q0, K)])

        pltpu.sync_copy(deg_l, deg_out.at[c, sid])

    return k(a_hbm, b_hbm, src, dst, wu, ovh)


def _sc_block2(a_hbm, b_hbm, src, dst, wu):
    @functools.partial(
        pl.kernel,
        out_type=jax.ShapeDtypeStruct((E, L), jnp.float32),
        mesh=_sc_mesh(),
        compiler_params=_sc_params(),
        scratch_types=[
            pltpu.VMEM((K,), jnp.int32),
            pltpu.VMEM((K,), jnp.int32),
            pltpu.VMEM((K,), jnp.int32),
            pltpu.VMEM((K,), jnp.int32),
            pltpu.VMEM((K, H), jnp.float32),
            pltpu.VMEM((K, H), jnp.float32),
            pltpu.VMEM((K, H), jnp.float32),
            pltpu.VMEM((K, H), jnp.float32),
            pltpu.VMEM((K, L), jnp.float32),
            pltpu.VMEM((H,), jnp.float32),
            pltpu.SemaphoreType.DMA,
            pltpu.SemaphoreType.DMA,
            pltpu.SemaphoreType.DMA,
            pltpu.SemaphoreType.DMA,
        ],
    )
    def k(a_ref, b_ref, src_ref, dst_ref, wu_ref,
          p_out, idx_sa, idx_da, idx_sb, idx_db, a_va, b_va, a_vb, b_vb,
          p_v, wu_v, sem_a, sem_b, sem_c, sem_d):
        c = lax.axis_index("c")
        sid = lax.axis_index("s")
        wid = sid * NC + c
        base = wid * EPW

        pltpu.sync_copy(wu_ref, wu_v)

        @pl.loop(0, CH // 2)
        def _(g):
            offa = base + 2 * g * K
            offb = offa + K
            pltpu.sync_copy(src_ref.at[pl.ds(offa, K)], idx_sa)
            pltpu.sync_copy(dst_ref.at[pl.ds(offa, K)], idx_da)
            cpa1 = pltpu.async_copy(a_ref.at[idx_sa], a_va, sem_a)
            cpa2 = pltpu.async_copy(b_ref.at[idx_da], b_va, sem_b)
            pltpu.sync_copy(src_ref.at[pl.ds(offb, K)], idx_sb)
            pltpu.sync_copy(dst_ref.at[pl.ds(offb, K)], idx_db)
            cpb1 = pltpu.async_copy(a_ref.at[idx_sb], a_vb, sem_c)
            cpb2 = pltpu.async_copy(b_ref.at[idx_db], b_vb, sem_d)
            cpa1.wait()
            cpa2.wait()
            _edge_dot(a_va, b_va, wu_v, p_v)
            pltpu.sync_copy(p_v, p_out.at[pl.ds(offa, K)])
            cpb1.wait()
            cpb2.wait()
            _edge_dot(a_vb, b_vb, wu_v, p_v)
            pltpu.sync_copy(p_v, p_out.at[pl.ds(offb, K)])

    return k(a_hbm, b_hbm, src, dst, wu)


def kernel(features, emb, edge_index, W_z, b_z, W_l, b_l, W_u, b_u):
    src = edge_index[0]
    dst = edge_index[1]
    w1t = W_l[:, :H].T
    w2t = W_l[:, H:].T
    bl = b_l.reshape(1, H)
    wu = W_u[0]

    a0, b_tab = _tc_prep(emb, features, w1t, w2t, bl)

    ovh = jnp.concatenate(
        [jnp.zeros((8,), jnp.float32), jnp.ones((8,), jnp.float32)]
    )

    p1, asum, degp = _sc_block1(a0, b_tab, src, dst, wu, ovh)
    dparts_t = degp.reshape(NW, NP).T[:N]
    a1 = _tc_update(a0, asum[0, :N], asum[1, :N], dparts_t)
    p2 = _sc_block2(a1, b_tab, src, dst, wu)

    s1, s2 = _tc_finish(p1, p2, b_u.reshape(1, 1))
    return (s1, s2)
